# baseline (speedup 1.0000x reference)
import functools

import jax
import jax.numpy as jnp
from jax import lax
from jax.experimental import pallas as pl
from jax.experimental.pallas import tpu as pltpu
from jax.experimental.pallas import tpu_sc as plsc


def _build_scatter(B, N_old, F, N_new):
    info = plsc.get_sparse_core_info()
    NC, NS, L = info.num_cores, info.num_subcores, info.num_lanes
    assert B % NC == 0 and N_old % NS == 0 and N_new % NS == 0 and F % L == 0
    BPC = B // NC
    RPT = N_old // NS
    OPT = N_new // NS
    ZR = 64
    assert OPT % ZR == 0

    mesh = plsc.VectorSubcoreMesh(core_axis_name="c", subcore_axis_name="s")

    @functools.partial(
        pl.kernel,
        mesh=mesh,
        out_type=jax.ShapeDtypeStruct((B * N_new, F), jnp.float32),
        compiler_params=pltpu.CompilerParams(use_tc_tiling_on_sc=False),
        scratch_types=[
            pltpu.VMEM_SHARED((N_new, F), jnp.float32),
            pltpu.VMEM((ZR, F), jnp.float32),
            pltpu.VMEM((RPT,), jnp.int32),
            pltpu.VMEM((RPT, F), jnp.float32),
        ],
    )
    def scatter_kernel(x_hbm, idx_hbm, out_hbm, acc, zbuf, idxv, rowsv):
        c = lax.axis_index("c")
        s = lax.axis_index("s")
        zv = jnp.zeros((L,), jnp.float32)

        def zstore(k, _):
            r = k // (F // L)
            off = (k % (F // L)) * L
            zbuf[r, pl.ds(off, L)] = zv
            return 0
        lax.fori_loop(0, ZR * (F // L), zstore, 0)

        for p in range(BPC):
            b = c * BPC + p
            for j in range(OPT // ZR):
                pltpu.sync_copy(zbuf, acc.at[pl.ds(s * OPT + j * ZR, ZR)])
            plsc.subcore_barrier()
            pltpu.sync_copy(idx_hbm.at[b, pl.ds(s * RPT, RPT)], idxv)
            pltpu.sync_copy(x_hbm.at[pl.ds(b * N_old + s * RPT, RPT)], rowsv)
            pltpu.sync_copy(rowsv, acc.at[idxv.at[:]], add=True)
            plsc.subcore_barrier()
            pltpu.sync_copy(
                acc.at[pl.ds(s * OPT, OPT)],
                out_hbm.at[pl.ds(b * N_new + s * OPT, OPT)],
            )

    return scatter_kernel


def _build_copy(shape, dtype, nchunk):
    rows = shape[0]
    assert rows % nchunk == 0
    crows = rows // nchunk

    def copy_body(a_hbm, out_hbm, sem):
        for i in range(nchunk):
            pltpu.make_async_copy(
                a_hbm.at[pl.ds(i * crows, crows)],
                out_hbm.at[pl.ds(i * crows, crows)],
                sem,
            ).start()
        for i in range(nchunk):
            pltpu.make_async_copy(
                a_hbm.at[pl.ds(i * crows, crows)],
                out_hbm.at[pl.ds(i * crows, crows)],
                sem,
            ).wait()

    return pl.pallas_call(
        copy_body,
        in_specs=[pl.BlockSpec(memory_space=pl.ANY)],
        out_specs=pl.BlockSpec(memory_space=pl.ANY),
        out_shape=jax.ShapeDtypeStruct(shape, dtype),
        scratch_shapes=[pltpu.SemaphoreType.DMA],
    )


def kernel(X, A, idx):
    B, N_old, F = X.shape
    N_new = A.shape[1]
    X_flat = X.reshape(B * N_old, F)
    idx2 = idx.reshape(B, N_old)
    new_X = _build_scatter(B, N_old, F, N_new)(X_flat, idx2)
    A2 = A.reshape(B * N_new, N_new)
    A_out = _build_copy(A2.shape, A2.dtype, 16)(A2).reshape(A.shape)
    return (new_X.reshape(B, N_new, F), A_out)


# speedup vs baseline: 37.1141x; 37.1141x over previous
import functools

import jax
import jax.numpy as jnp
from jax import lax
from jax.experimental import pallas as pl
from jax.experimental.pallas import tpu as pltpu
from jax.experimental.pallas import tpu_sc as plsc


def _build_scatter(B, N_old, F, N_new):
    info = plsc.get_sparse_core_info()
    NC, NS, L = info.num_cores, info.num_subcores, info.num_lanes
    assert B % NC == 0 and N_old % NS == 0 and N_new % NS == 0 and F % L == 0
    BPC = B // NC
    RPT = N_old // NS
    OPT = N_new // NS
    ZR = 64
    assert OPT % ZR == 0

    mesh = plsc.VectorSubcoreMesh(core_axis_name="c", subcore_axis_name="s")

    @functools.partial(
        pl.kernel,
        mesh=mesh,
        out_type=jax.ShapeDtypeStruct((B * N_new, F), jnp.float32),
        compiler_params=pltpu.CompilerParams(use_tc_tiling_on_sc=False),
        scratch_types=[
            pltpu.VMEM_SHARED((N_new, F), jnp.float32),
            pltpu.VMEM((ZR, F), jnp.float32),
            pltpu.VMEM((RPT,), jnp.int32),
            pltpu.VMEM((RPT, F), jnp.float32),
        ],
    )
    def scatter_kernel(x_hbm, idx_hbm, out_hbm, acc, zbuf, idxv, rowsv):
        c = lax.axis_index("c")
        s = lax.axis_index("s")
        zv = jnp.zeros((L,), jnp.float32)

        def zstore(k, _):
            r = k // (F // L)
            off = (k % (F // L)) * L
            zbuf[r, pl.ds(off, L)] = zv
            return 0
        lax.fori_loop(0, ZR * (F // L), zstore, 0)

        for p in range(BPC):
            b = c * BPC + p
            for j in range(OPT // ZR):
                pltpu.sync_copy(zbuf, acc.at[pl.ds(s * OPT + j * ZR, ZR)])
            plsc.subcore_barrier()
            pltpu.sync_copy(idx_hbm.at[b, pl.ds(s * RPT, RPT)], idxv)
            pltpu.sync_copy(x_hbm.at[pl.ds(b * N_old + s * RPT, RPT)], rowsv)
            pltpu.sync_copy(rowsv, acc.at[idxv.at[:]], add=True)
            plsc.subcore_barrier()
            pltpu.sync_copy(
                acc.at[pl.ds(s * OPT, OPT)],
                out_hbm.at[pl.ds(b * N_new + s * OPT, OPT)],
            )

    return scatter_kernel


def _build_copy(shape, dtype, block_rows):
    rows, cols = shape
    assert rows % block_rows == 0

    def copy_body(a_ref, out_ref):
        out_ref[...] = a_ref[...]

    return pl.pallas_call(
        copy_body,
        grid=(rows // block_rows,),
        in_specs=[pl.BlockSpec((block_rows, cols), lambda i: (i, 0))],
        out_specs=pl.BlockSpec((block_rows, cols), lambda i: (i, 0)),
        out_shape=jax.ShapeDtypeStruct(shape, dtype),
    )


def kernel(X, A, idx):
    B, N_old, F = X.shape
    N_new = A.shape[1]
    X_flat = X.reshape(B * N_old, F)
    idx2 = idx.reshape(B, N_old)
    new_X = _build_scatter(B, N_old, F, N_new)(X_flat, idx2)
    A2 = A.reshape(B * N_new, N_new)
    A_out = _build_copy(A2.shape, A2.dtype, 512)(A2).reshape(A.shape)
    return (new_X.reshape(B, N_new, F), A_out)


# speedup vs baseline: 40.8062x; 1.0995x over previous
import functools

import jax
import jax.numpy as jnp
from jax import lax
from jax.experimental import pallas as pl
from jax.experimental.pallas import tpu as pltpu
from jax.experimental.pallas import tpu_sc as plsc


def _build_scatter(B, N_old, F, N_new):
    info = plsc.get_sparse_core_info()
    NC, NS, L = info.num_cores, info.num_subcores, info.num_lanes
    assert B % NC == 0 and N_old % NS == 0 and N_new % NS == 0 and F % L == 0
    BPC = B // NC
    RPT = N_old // NS
    OPT = N_new // NS
    ZR = 64
    assert OPT % ZR == 0

    mesh = plsc.VectorSubcoreMesh(core_axis_name="c", subcore_axis_name="s")

    TCOLS = F // 128
    @functools.partial(
        pl.kernel,
        mesh=mesh,
        out_type=jax.ShapeDtypeStruct((B * N_new // 8, TCOLS, 8, 128), jnp.float32),
        compiler_params=pltpu.CompilerParams(use_tc_tiling_on_sc=False),
        scratch_types=[
            pltpu.VMEM_SHARED((N_new, F), jnp.float32),
            pltpu.VMEM((ZR, F), jnp.float32),
            pltpu.VMEM((RPT,), jnp.int32),
            pltpu.VMEM((RPT, F), jnp.float32),
            pltpu.SemaphoreType.DMA,
        ],
    )
    def scatter_kernel(x_hbm, idx_hbm, out_hbm, acc, zbuf, idxv, rowsv, sem):
        c = lax.axis_index("c")
        s = lax.axis_index("s")
        zv = jnp.zeros((L,), jnp.float32)

        def zstore(k, _):
            r = k // (F // L)
            off = (k % (F // L)) * L
            zbuf[r, pl.ds(off, L)] = zv
            return 0
        lax.fori_loop(0, ZR * (F // L), zstore, 0)

        for p in range(BPC):
            b = c * BPC + p
            for j in range(OPT // ZR):
                pltpu.sync_copy(zbuf, acc.at[pl.ds(s * OPT + j * ZR, ZR)])
            plsc.subcore_barrier()
            pltpu.sync_copy(idx_hbm.at[b, pl.ds(s * RPT, RPT)], idxv)
            pltpu.sync_copy(x_hbm.at[pl.ds(b * N_old + s * RPT, RPT)], rowsv)
            pltpu.sync_copy(rowsv, acc.at[idxv.at[:]], add=True)
            plsc.subcore_barrier()
            copies = []
            for g in range(OPT // 8):
                t0 = s * OPT + g * 8
                for tc in range(TCOLS):
                    copies.append(pltpu.make_async_copy(
                        acc.at[pl.ds(t0, 8), pl.ds(tc * 128, 128)],
                        out_hbm.at[(b * N_new + t0) // 8, tc],
                        sem,
                    ))
            for cp in copies:
                cp.start()
            for cp in copies:
                cp.wait()

    return scatter_kernel


def _build_copy(shape, dtype, block_rows):
    rows, cols = shape
    assert rows % block_rows == 0

    def copy_body(a_ref, out_ref):
        out_ref[...] = a_ref[...]

    return pl.pallas_call(
        copy_body,
        grid=(rows // block_rows,),
        in_specs=[pl.BlockSpec((block_rows, cols), lambda i: (i, 0))],
        out_specs=pl.BlockSpec((block_rows, cols), lambda i: (i, 0)),
        out_shape=jax.ShapeDtypeStruct(shape, dtype),
    )


def kernel(X, A, idx):
    B, N_old, F = X.shape
    N_new = A.shape[1]
    X_flat = X.reshape(B * N_old, F)
    idx2 = idx.reshape(B, N_old)
    out4 = _build_scatter(B, N_old, F, N_new)(X_flat, idx2)
    new_X = out4.transpose(0, 2, 1, 3).reshape(B, N_new, F)
    A2 = A.reshape(B * N_new, N_new)
    A_out = _build_copy(A2.shape, A2.dtype, 512)(A2).reshape(A.shape)
    return (new_X, A_out)


# speedup vs baseline: 43.0506x; 1.0550x over previous
import functools

import jax
import jax.numpy as jnp
from jax import lax
from jax.experimental import pallas as pl
from jax.experimental.pallas import tpu as pltpu
from jax.experimental.pallas import tpu_sc as plsc


def _build_scatter(B, N_old, F, N_new):
    info = plsc.get_sparse_core_info()
    NC, NS, L = info.num_cores, info.num_subcores, info.num_lanes
    assert B % NC == 0 and N_old % NS == 0 and N_new % NS == 0 and F == 256
    assert L == 16
    BPC = B // NC
    RPT = N_old // NS
    OPT = N_new // NS
    ZR = 64
    HPT = 2 * RPT
    assert (2 * OPT) % ZR == 0 and RPT % L == 0

    mesh = plsc.VectorSubcoreMesh(core_axis_name="c", subcore_axis_name="s")

    @functools.partial(
        pl.kernel,
        mesh=mesh,
        out_type=jax.ShapeDtypeStruct((B * N_new // 8, 2, 8, 128), jnp.float32),
        compiler_params=pltpu.CompilerParams(use_tc_tiling_on_sc=False),
        scratch_types=[
            pltpu.VMEM_SHARED((2 * N_new, 128), jnp.float32),
            pltpu.VMEM((ZR, 128), jnp.float32),
            pltpu.VMEM((RPT,), jnp.int32),
            pltpu.VMEM((RPT,), jnp.int32),
            pltpu.VMEM((RPT,), jnp.int32),
            pltpu.VMEM((HPT, 128), jnp.float32),
            pltpu.SemaphoreType.DMA,
        ],
    )
    def scatter_kernel(x_hbm, idx_hbm, out_hbm, acc, zbuf, idxv, jla, jlb,
                       rowsv, sem):
        c = lax.axis_index("c")
        s = lax.axis_index("s")
        zv = jnp.zeros((L,), jnp.float32)

        def zstore(k, _):
            r = k // (128 // L)
            off = (k % (128 // L)) * L
            zbuf[r, pl.ds(off, L)] = zv
            return 0
        lax.fori_loop(0, ZR * (128 // L), zstore, 0)

        iota = lax.iota(jnp.int32, L)
        perm_lo = iota & 7
        perm_hi = perm_lo + 8
        offs = jnp.where(iota < 8, 0, N_new)

        for p in range(BPC):
            b = c * BPC + p
            for j in range((2 * OPT) // ZR):
                pltpu.sync_copy(zbuf, acc.at[pl.ds(s * 2 * OPT + j * ZR, ZR)])
            tr0 = b * (N_old // 8) + s * (RPT // 8)
            stages = []
            for g in range(RPT // 8):
                for tc in range(2):
                    stages.append(pltpu.make_async_copy(
                        x_hbm.at[tr0 + g, tc],
                        rowsv.at[pl.ds((g * 2 + tc) * 8, 8)],
                        sem,
                    ))
            for cp in stages:
                cp.start()
            pltpu.sync_copy(idx_hbm.at[pl.ds(b * N_old + s * RPT, RPT)], idxv)
            for m in range(RPT // L):
                v = idxv[pl.ds(m * L, L)]
                glo = v[perm_lo] + offs
                ghi = v[perm_hi] + offs
                k0 = 2 * m * L
                if k0 + L <= RPT:
                    jla[pl.ds(k0, L)] = glo
                else:
                    jlb[pl.ds(k0 - RPT, L)] = glo
                k1 = k0 + L
                if k1 + L <= RPT:
                    jla[pl.ds(k1, L)] = ghi
                else:
                    jlb[pl.ds(k1 - RPT, L)] = ghi
            for cp in stages:
                cp.wait()
            plsc.subcore_barrier()
            pltpu.sync_copy(rowsv.at[pl.ds(0, RPT)], acc.at[jla.at[:]],
                            add=True)
            pltpu.sync_copy(rowsv.at[pl.ds(RPT, RPT)], acc.at[jlb.at[:]],
                            add=True)
            plsc.subcore_barrier()
            copies = []
            for g in range(OPT // 8):
                t0 = s * OPT + g * 8
                for tc in range(2):
                    copies.append(pltpu.make_async_copy(
                        acc.at[pl.ds(tc * N_new + t0, 8)],
                        out_hbm.at[(b * N_new + t0) // 8, tc],
                        sem,
                    ))
            for cp in copies:
                cp.start()
            for cp in copies:
                cp.wait()

    return scatter_kernel


def _build_copy(shape, dtype, block_rows):
    rows, cols = shape
    assert rows % block_rows == 0

    def copy_body(a_ref, out_ref):
        out_ref[...] = a_ref[...]

    return pl.pallas_call(
        copy_body,
        grid=(rows // block_rows,),
        in_specs=[pl.BlockSpec((block_rows, cols), lambda i: (i, 0))],
        out_specs=pl.BlockSpec((block_rows, cols), lambda i: (i, 0)),
        out_shape=jax.ShapeDtypeStruct(shape, dtype),
    )


def kernel(X, A, idx):
    B, N_old, F = X.shape
    N_new = A.shape[1]
    X4 = X.reshape(B * N_old // 8, 8, F // 128, 128).transpose(0, 2, 1, 3)
    idx1 = idx.reshape(B * N_old)
    out4 = _build_scatter(B, N_old, F, N_new)(X4, idx1)
    new_X = out4.transpose(0, 2, 1, 3).reshape(B, N_new, F)
    A2 = A.reshape(B * N_new, N_new)
    A_out = _build_copy(A2.shape, A2.dtype, 512)(A2).reshape(A.shape)
    return (new_X, A_out)


# speedup vs baseline: 43.0917x; 1.0010x over previous
import functools

import jax
import jax.numpy as jnp
from jax import lax
from jax.experimental import pallas as pl
from jax.experimental.pallas import tpu as pltpu
from jax.experimental.pallas import tpu_sc as plsc


def _build_scatter(B, N_old, F, N_new):
    info = plsc.get_sparse_core_info()
    NC, NS, L = info.num_cores, info.num_subcores, info.num_lanes
    assert B % NC == 0 and N_old % NS == 0 and N_new % NS == 0 and F == 256
    assert L == 16
    BPC = B // NC
    RPT = N_old // NS
    OPT = N_new // NS
    ZR = 64
    HPT = 2 * RPT
    assert (2 * OPT) % ZR == 0 and RPT % L == 0

    mesh = plsc.VectorSubcoreMesh(core_axis_name="c", subcore_axis_name="s")

    @functools.partial(
        pl.kernel,
        mesh=mesh,
        out_type=jax.ShapeDtypeStruct((B * N_new // 8, 2, 8, 128), jnp.float32),
        compiler_params=pltpu.CompilerParams(use_tc_tiling_on_sc=False),
        scratch_types=[
            pltpu.VMEM_SHARED((2 * N_new, 128), jnp.float32),
            pltpu.VMEM((ZR, 128), jnp.float32),
            pltpu.VMEM((RPT,), jnp.int32),
            pltpu.VMEM((RPT,), jnp.int32),
            pltpu.VMEM((RPT,), jnp.int32),
            pltpu.VMEM((HPT, 128), jnp.float32),
            pltpu.SemaphoreType.DMA,
        ],
    )
    def scatter_kernel(x_hbm, idx_hbm, out_hbm, acc, zbuf, idxv, jla, jlb,
                       rowsv, sem):
        c = lax.axis_index("c")
        s = lax.axis_index("s")
        zv = jnp.zeros((L,), jnp.float32)

        def zstore(k, _):
            r = k // (128 // L)
            off = (k % (128 // L)) * L
            zbuf[r, pl.ds(off, L)] = zv
            return 0
        lax.fori_loop(0, ZR * (128 // L), zstore, 0)

        iota = lax.iota(jnp.int32, L)
        perm_lo = iota & 7
        perm_hi = perm_lo + 8
        offs = jnp.where(iota < 8, 0, N_new)

        for p in range(BPC):
            b = c * BPC + p
            for tc in range(2):
                for j in range(OPT // ZR):
                    pltpu.sync_copy(
                        zbuf,
                        acc.at[pl.ds(tc * N_new + s * OPT + j * ZR, ZR)],
                    )
            tr0 = b * (N_old // 8) + s * (RPT // 8)
            stages = []
            for g in range(RPT // 8):
                for tc in range(2):
                    stages.append(pltpu.make_async_copy(
                        x_hbm.at[tr0 + g, tc],
                        rowsv.at[pl.ds((g * 2 + tc) * 8, 8)],
                        sem,
                    ))
            for cp in stages:
                cp.start()
            pltpu.sync_copy(idx_hbm.at[pl.ds(b * N_old + s * RPT, RPT)], idxv)
            for m in range(RPT // L):
                v = idxv[pl.ds(m * L, L)]
                glo = v[perm_lo] + offs
                ghi = v[perm_hi] + offs
                k0 = 2 * m * L
                if k0 + L <= RPT:
                    jla[pl.ds(k0, L)] = glo
                else:
                    jlb[pl.ds(k0 - RPT, L)] = glo
                k1 = k0 + L
                if k1 + L <= RPT:
                    jla[pl.ds(k1, L)] = ghi
                else:
                    jlb[pl.ds(k1 - RPT, L)] = ghi
            for cp in stages:
                cp.wait()
            plsc.subcore_barrier()
            pltpu.sync_copy(rowsv.at[pl.ds(0, RPT)], acc.at[jla.at[:]],
                            add=True)
            pltpu.sync_copy(rowsv.at[pl.ds(RPT, RPT)], acc.at[jlb.at[:]],
                            add=True)
            plsc.subcore_barrier()
            copies = []
            for g in range(OPT // 8):
                t0 = s * OPT + g * 8
                for tc in range(2):
                    copies.append(pltpu.make_async_copy(
                        acc.at[pl.ds(tc * N_new + t0, 8)],
                        out_hbm.at[(b * N_new + t0) // 8, tc],
                        sem,
                    ))
            for cp in copies:
                cp.start()
            for cp in copies:
                cp.wait()

    return scatter_kernel


def _build_copy(shape, dtype, block_rows):
    rows, cols = shape
    assert rows % block_rows == 0

    def copy_body(a_ref, out_ref):
        out_ref[...] = a_ref[...]

    return pl.pallas_call(
        copy_body,
        grid=(rows // block_rows,),
        in_specs=[pl.BlockSpec((block_rows, cols), lambda i: (i, 0))],
        out_specs=pl.BlockSpec((block_rows, cols), lambda i: (i, 0)),
        out_shape=jax.ShapeDtypeStruct(shape, dtype),
        compiler_params=pltpu.CompilerParams(vmem_limit_bytes=100_000_000),
    )


def kernel(X, A, idx):
    B, N_old, F = X.shape
    N_new = A.shape[1]
    X4 = X.reshape(B * N_old // 8, 8, F // 128, 128).transpose(0, 2, 1, 3)
    idx1 = idx.reshape(B * N_old)
    out4 = _build_scatter(B, N_old, F, N_new)(X4, idx1)
    new_X = out4.transpose(0, 2, 1, 3).reshape(B, N_new, F)
    A2 = A.reshape(B * N_new, N_new)
    A_out = _build_copy(A2.shape, A2.dtype, 512)(A2).reshape(A.shape)
    return (new_X, A_out)
